# baseline (device time: 125884 ns/iter reference)
import jax
import jax.numpy as jnp
from jax import lax
from jax.experimental import pallas as pl
from jax.experimental.pallas import tpu as pltpu

N_DEV = 4
N_TOK = 2048
D_MODEL = 1024
E_GLOBAL = 32
E_LOCAL = E_GLOBAL // N_DEV
COLS = 256
N_CHUNK = D_MODEL // COLS
HALF = N_TOK // 2


def kernel(x, router_W, route_idx, expert_W):
    def body(x_ref, rw_ref, idx_ref, ew_hbm, out_ref,
             ew_buf, ew_sems, ex_send, ex_recv, send_sems, recv_sems):
        my_pos = lax.axis_index("i")
        left = lax.rem(my_pos - 1 + N_DEV, N_DEV)
        right = lax.rem(my_pos + 1, N_DEV)
        even = lax.rem(my_pos, 2) == 0
        pair_a = jnp.where(even, right, left)
        pair_b = jnp.where(even, left, right)

        barrier_sem = pltpu.get_barrier_semaphore()
        for nbr in [left, right]:
            pl.semaphore_signal(
                barrier_sem, inc=1,
                device_id=(nbr,), device_id_type=pl.DeviceIdType.MESH,
            )
        pl.semaphore_wait(barrier_sem, 2)

        def w_copy(j):
            c, e = divmod(j, E_LOCAL)
            return pltpu.make_async_copy(
                ew_hbm.at[e, :, c * COLS:(c + 1) * COLS],
                ew_buf.at[j % 2], ew_sems.at[j % 2])

        w_copy(0).start()

        xv = x_ref[:, :]

        scores = jnp.dot(xv, rw_ref[:, :], preferred_element_type=jnp.float32)
        s_max = jnp.max(scores, axis=-1, keepdims=True)
        p = jnp.exp(scores - s_max)
        probs = p / jnp.sum(p, axis=-1, keepdims=True)

        idx = idx_ref[:, :]
        e_ids = lax.broadcasted_iota(jnp.int32, (N_TOK, E_GLOBAL), 1)
        g0 = jnp.sum(jnp.where(e_ids == idx[:, 0:1], probs, 0.0), axis=-1,
                     keepdims=True)
        g1 = jnp.sum(jnp.where(e_ids == idx[:, 1:2], probs, 0.0), axis=-1,
                     keepdims=True)
        gs = g0 + g1
        w0 = g0 / gs
        w1 = g1 / gs

        def gate_for(e_glob):
            return (jnp.where(idx[:, 0:1] == e_glob, w0, 0.0)
                    + jnp.where(idx[:, 1:2] == e_glob, w1, 0.0))

        def rows(half):
            return pl.ds(half * HALF, HALF)

        def col(c):
            return pl.ds(c * COLS, COLS)

        def partner(stage, half):
            return pair_a if (stage + half) % 2 == 0 else pair_b

        def exchange_start(stage, c):
            q = c % 2
            for half in (0, 1):
                ex_send[stage, half, q, :, :] = out_ref[
                    rows(half), col(c)].astype(jnp.bfloat16)
                rdma = pltpu.make_async_remote_copy(
                    src_ref=ex_send.at[stage, half, q],
                    dst_ref=ex_recv.at[stage, half, q],
                    send_sem=send_sems.at[stage, half, q],
                    recv_sem=recv_sems.at[stage, half, q],
                    device_id=(partner(stage, half),),
                    device_id_type=pl.DeviceIdType.MESH,
                )
                rdma.start()

        def exchange_finish(stage, c):
            q = c % 2
            for half in (0, 1):
                rdma = pltpu.make_async_remote_copy(
                    src_ref=ex_send.at[stage, half, q],
                    dst_ref=ex_recv.at[stage, half, q],
                    send_sem=send_sems.at[stage, half, q],
                    recv_sem=recv_sems.at[stage, half, q],
                    device_id=(partner(stage, half),),
                    device_id_type=pl.DeviceIdType.MESH,
                )
                rdma.wait()
                out_ref[rows(half), col(c)] = out_ref[
                    rows(half), col(c)] + ex_recv[
                    stage, half, q, :, :].astype(jnp.float32)

        for c in range(N_CHUNK):
            acc = jnp.zeros((N_TOK, COLS), dtype=jnp.float32)
            for e in range(E_LOCAL):
                j = c * E_LOCAL + e
                if j + 1 < N_CHUNK * E_LOCAL:
                    w_copy(j + 1).start()
                w_copy(j).wait()
                d = jnp.dot(xv, ew_buf[j % 2],
                            preferred_element_type=jnp.float32)
                acc = acc + gate_for(my_pos * E_LOCAL + e) * d
            out_ref[:, col(c)] = acc

            if c >= 2:
                exchange_finish(1, c - 2)
            if c >= 1:
                exchange_finish(0, c - 1)
                exchange_start(1, c - 1)
            exchange_start(0, c)

        exchange_finish(1, N_CHUNK - 2)
        exchange_finish(0, N_CHUNK - 1)
        exchange_start(1, N_CHUNK - 1)
        exchange_finish(1, N_CHUNK - 1)

    return pl.pallas_call(
        body,
        out_shape=jax.ShapeDtypeStruct((N_TOK, D_MODEL), jnp.float32),
        in_specs=[
            pl.BlockSpec(memory_space=pltpu.VMEM),
            pl.BlockSpec(memory_space=pltpu.VMEM),
            pl.BlockSpec(memory_space=pltpu.VMEM),
            pl.BlockSpec(memory_space=pltpu.MemorySpace.HBM),
        ],
        out_specs=pl.BlockSpec(memory_space=pltpu.VMEM),
        scratch_shapes=[
            pltpu.VMEM((2, D_MODEL, COLS), jnp.float32),
            pltpu.SemaphoreType.DMA((2,)),
            pltpu.VMEM((2, 2, 2, HALF, COLS), jnp.bfloat16),
            pltpu.VMEM((2, 2, 2, HALF, COLS), jnp.bfloat16),
            pltpu.SemaphoreType.DMA((2, 2, 2)),
            pltpu.SemaphoreType.DMA((2, 2, 2)),
        ],
        compiler_params=pltpu.CompilerParams(
            collective_id=0, vmem_limit_bytes=60 * 1024 * 1024),
    )(x, router_W, route_idx, expert_W)
